# baseline (speedup 1.0000x reference)
import functools

import jax
import jax.numpy as jnp
from jax.experimental import pallas as pl
from jax.experimental.pallas import tpu as pltpu

B, N, D = 16, 4096, 256
K = N // 2

SCORE_TILE = 1024


def _score_body(x_ref, p0_ref, s_ref):
    s_ref[...] = jnp.dot(x_ref[...], p0_ref[...],
                         preferred_element_type=jnp.float32)


def _score(x2, p0):
    grid = (B * N // SCORE_TILE,)
    return pl.pallas_call(
        _score_body,
        grid=grid,
        in_specs=[
            pl.BlockSpec((SCORE_TILE, D), lambda i: (i, 0)),
            pl.BlockSpec((D, 1), lambda i: (0, 0)),
        ],
        out_specs=pl.BlockSpec((SCORE_TILE, 1), lambda i: (i, 0)),
        out_shape=jax.ShapeDtypeStruct((B * N, 1), jnp.float32),
    )(x2, p0)


GATE_TILE = 512


def _gate_body(xb_ref, p_ref, norm_ref, o_ref):
    xb = xb_ref[...]
    y = jnp.dot(xb, p_ref[...], preferred_element_type=jnp.float32)
    y = y / norm_ref[0]
    o_ref[...] = xb * jax.nn.sigmoid(y)


def _gate(x_bar2, p, norm):
    grid = (B * K // GATE_TILE,)
    return pl.pallas_call(
        _gate_body,
        grid=grid,
        in_specs=[
            pl.BlockSpec((GATE_TILE, D), lambda i: (i, 0)),
            pl.BlockSpec((D, D), lambda i: (0, 0)),
            pl.BlockSpec(memory_space=pltpu.SMEM),
        ],
        out_specs=pl.BlockSpec((GATE_TILE, D), lambda i: (i, 0)),
        out_shape=jax.ShapeDtypeStruct((B * K, D), jnp.float32),
    )(x_bar2, p, norm)


@jax.jit
def kernel(x, p):
    norm = jnp.sqrt(jnp.sum(p ** 2)).reshape(1)
    p0 = p[:, 0].reshape(D, 1)
    score = _score(x.reshape(B * N, D), p0).reshape(B, N) / norm
    _, top_idx = jax.lax.top_k(score, K)
    x_bar = jnp.take_along_axis(x, top_idx[:, :, None], axis=1)
    out = _gate(x_bar.reshape(B * K, D), p, norm)
    return out.reshape(B, K, D), top_idx
